# baseline (speedup 1.0000x reference)
import functools

import jax
import jax.numpy as jnp
from jax import lax
from jax.experimental import pallas as pl
from jax.experimental.pallas import tpu as pltpu
import jax.experimental.pallas.tpu_sc as plsc

N = 10000
D = 128
E = 320000
NC = 2
NS = 16
NW = NC * NS
EPW = E // NW
CHUNK = 128
NCH = 80
NCHP = NCH + 2
EPWP = NCHP * CHUNK
NACC = 10240
RPT = NACC // NS
ZCH = RPT // CHUNK
JUNK = N + 16


def _sc_mesh():
    return plsc.VectorSubcoreMesh(core_axis_name="c", subcore_axis_name="s")


def _deg_pass(dst_3d):
    @functools.partial(
        pl.kernel,
        out_type=jax.ShapeDtypeStruct((NC * NACC,), jnp.float32),
        mesh=_sc_mesh(),
        scratch_types=[
            pltpu.VMEM((NCHP, CHUNK), jnp.int32),
            pltpu.VMEM((CHUNK,), jnp.float32),
            pltpu.VMEM((CHUNK,), jnp.float32),
            pltpu.VMEM_SHARED((NACC,), jnp.float32),
            pltpu.SemaphoreType.DMA,
        ],
    )
    def deg_kernel(dst_hbm, out_hbm, dstv, onev, zerov, dacc, ssem):
        c = lax.axis_index("c")
        s = lax.axis_index("s")
        w = c * NS + s
        for j in range(CHUNK // 16):
            onev[pl.ds(j * 16, 16)] = jnp.ones((16,), jnp.float32)
            zerov[pl.ds(j * 16, 16)] = jnp.zeros((16,), jnp.float32)
        for i in range(ZCH):
            pltpu.sync_copy(zerov, dacc.at[pl.ds(s * RPT + i * CHUNK, CHUNK)])
        pltpu.sync_copy(dst_hbm.at[w], dstv)
        plsc.subcore_barrier()

        def body(k, carry):
            pltpu.sync_copy(onev, dacc.at[dstv.at[k]], add=True)
            return carry

        lax.fori_loop(0, NCH, body, 0)
        plsc.subcore_barrier()
        pltpu.sync_copy(dacc.at[pl.ds(s * RPT, RPT)],
                        out_hbm.at[pl.ds(c * NACC + s * RPT, RPT)])

    return deg_kernel(dst_3d)


def _edge_pass(g, src_flat, dst_flat):
    @functools.partial(
        pl.kernel,
        out_type=jax.ShapeDtypeStruct((NC * NACC, D), jnp.float32),
        mesh=_sc_mesh(),
        scratch_types=[
            pltpu.VMEM((CHUNK,), jnp.int32),
            pltpu.VMEM((CHUNK,), jnp.int32),
            pltpu.VMEM((CHUNK,), jnp.int32),
            pltpu.VMEM((CHUNK,), jnp.int32),
            pltpu.VMEM((CHUNK, D), jnp.float32),
            pltpu.VMEM((CHUNK, D), jnp.float32),
            pltpu.VMEM_SHARED((NACC, D), jnp.float32),
            pltpu.SemaphoreType.DMA,
            pltpu.SemaphoreType.DMA,
            pltpu.SemaphoreType.DMA,
            pltpu.SemaphoreType.DMA,
        ],
    )
    def edge_kernel(g_hbm, src_hbm, dst_hbm, out_hbm,
                    sv0, sv1, dv0, dv1, r0, r1, acc, gs0, gs1, is0, is1):
        srcb = [sv0, sv1]
        dstb = [dv0, dv1]
        rows = [r0, r1]
        gsem = [gs0, gs1]
        isem = [is0, is1]
        c = lax.axis_index("c")
        s = lax.axis_index("s")
        w = c * NS + s

        def zero_row(r, carry):
            for j in range(D // 16):
                rows[0][r, pl.ds(j * 16, 16)] = jnp.zeros((16,), jnp.float32)
            return carry

        lax.fori_loop(0, CHUNK, zero_row, 0)
        for i in range(ZCH):
            pltpu.sync_copy(rows[0], acc.at[pl.ds(s * RPT + i * CHUNK, CHUNK)])
        plsc.subcore_barrier()

        ebase = w * EPWP

        def fetch_idx(k, b, sem):
            pltpu.async_copy(src_hbm.at[pl.ds(ebase + k * CHUNK, CHUNK)],
                             srcb[b], sem)
            pltpu.async_copy(dst_hbm.at[pl.ds(ebase + k * CHUNK, CHUNK)],
                             dstb[b], sem)

        def wait_idx(k, b, sem):
            pltpu.make_async_copy(
                src_hbm.at[pl.ds(ebase + k * CHUNK, CHUNK)], srcb[b],
                sem).wait()
            pltpu.make_async_copy(
                dst_hbm.at[pl.ds(ebase + k * CHUNK, CHUNK)], dstb[b],
                sem).wait()

        fetch_idx(0, 0, isem[0])
        wait_idx(0, 0, isem[0])
        pltpu.async_copy(g_hbm.at[srcb[0]], rows[0], gsem[0])
        fetch_idx(1, 1, isem[1])

        def body(t, carry):
            k2 = 2 * t
            for b in range(2):
                k = k2 + b
                nb = 1 - b
                pltpu.make_async_copy(g_hbm.at[srcb[b]], rows[b],
                                      gsem[b]).wait()
                wait_idx(k + 1, nb, isem[nb])
                pltpu.async_copy(g_hbm.at[srcb[nb]], rows[nb], gsem[nb])
                pltpu.sync_copy(rows[b], acc.at[dstb[b]], add=True)
                fetch_idx(k + 2, b, isem[b])
            return carry

        lax.fori_loop(0, NCH // 2, body, 0)
        pltpu.make_async_copy(g_hbm.at[srcb[0]], rows[0], gsem[0]).wait()
        wait_idx(NCH + 1, 1, isem[1])
        plsc.subcore_barrier()
        pltpu.sync_copy(acc.at[pl.ds(s * RPT, RPT)],
                        out_hbm.at[pl.ds(c * NACC + s * RPT, RPT)])

    return edge_kernel(g, src_flat, dst_flat)


def _tc1_body(x_ref, w_ref, dinv_ref, g_ref):
    h = jnp.dot(x_ref[...], w_ref[...], preferred_element_type=jnp.float32)
    g_ref[...] = h * dinv_ref[...]


def _tc2_body(acc_ref, g_ref, dinv_ref, b_ref, w_ref, o_ref):
    accsum = acc_ref[0:N, :] + acc_ref[NACC:NACC + N, :]
    hf = jax.nn.relu((accsum + g_ref[...]) * dinv_ref[...] + b_ref[...])
    h2 = jnp.dot(hf, w_ref[...], preferred_element_type=jnp.float32)
    o_ref[...] = h2 * dinv_ref[...]


def _tc3_body(acc_ref, g_ref, dinv_ref, b_ref, wo_ref, bo_ref, o_ref):
    accsum = acc_ref[0:N, :] + acc_ref[NACC:NACC + N, :]
    hf = jax.nn.relu((accsum + g_ref[...]) * dinv_ref[...] + b_ref[...])
    z = jnp.dot(hf, wo_ref[...], preferred_element_type=jnp.float32)
    o_ref[...] = jax.nn.sigmoid(z + bo_ref[...])


def kernel(x, edge_index, W1, b1, W2, b2, Wo, bo):
    ei = edge_index.astype(jnp.int32)
    pad = ((0, 0), (0, EPWP - EPW))
    src_3d = jnp.pad(ei[0].reshape(NW, EPW), pad).reshape(NW, NCHP, CHUNK)
    dst_3d = jnp.pad(ei[1].reshape(NW, EPW), pad,
                     constant_values=JUNK).reshape(NW, NCHP, CHUNK)
    src_flat = src_3d.reshape(NW * EPWP)
    dst_flat = dst_3d.reshape(NW * EPWP)

    deg2 = _deg_pass(dst_3d)
    deg = deg2[:N] + deg2[NACC:NACC + N] + 1.0
    dinv = lax.rsqrt(deg)[:, None]

    g1 = pl.pallas_call(
        _tc1_body,
        out_shape=jax.ShapeDtypeStruct((N, D), jnp.float32),
    )(x, W1, dinv)

    acc1 = _edge_pass(g1, src_flat, dst_flat)

    g2 = pl.pallas_call(
        _tc2_body,
        out_shape=jax.ShapeDtypeStruct((N, D), jnp.float32),
    )(acc1, g1, dinv, b1.reshape(1, D), W2)

    acc2 = _edge_pass(g2, src_flat, dst_flat)

    out = pl.pallas_call(
        _tc3_body,
        out_shape=jax.ShapeDtypeStruct((N, 1), jnp.float32),
    )(acc2, g2, dinv, b2.reshape(1, D), Wo, bo.reshape(1, 1))
    return out


# speedup vs baseline: 1.2374x; 1.2374x over previous
import functools

import jax
import jax.numpy as jnp
from jax import lax
from jax.experimental import pallas as pl
from jax.experimental.pallas import tpu as pltpu
import jax.experimental.pallas.tpu_sc as plsc

N = 10000
D = 128
E = 320000
NC = 2
NS = 16
NW = NC * NS
EPW = E // NW
CHUNK = 128
CPC = 2
ECH = CPC * CHUNK
NCH = 40
NCHP = NCH
EPWP = NCH * ECH
NACC = 10240
RPT = NACC // NS
ZCH = RPT // CHUNK
JUNK = N + 16


def _sc_mesh():
    return plsc.VectorSubcoreMesh(core_axis_name="c", subcore_axis_name="s")


def _deg_pass(dst_3d):
    @functools.partial(
        pl.kernel,
        out_type=jax.ShapeDtypeStruct((NC * NACC,), jnp.float32),
        mesh=_sc_mesh(),
        scratch_types=[
            pltpu.VMEM((EPWP // CHUNK, CHUNK), jnp.int32),
            pltpu.VMEM((CHUNK,), jnp.float32),
            pltpu.VMEM((CHUNK,), jnp.float32),
            pltpu.VMEM_SHARED((NACC,), jnp.float32),
            pltpu.SemaphoreType.DMA,
        ],
    )
    def deg_kernel(dst_hbm, out_hbm, dstv, onev, zerov, dacc, ssem):
        c = lax.axis_index("c")
        s = lax.axis_index("s")
        w = c * NS + s
        for j in range(CHUNK // 16):
            onev[pl.ds(j * 16, 16)] = jnp.ones((16,), jnp.float32)
            zerov[pl.ds(j * 16, 16)] = jnp.zeros((16,), jnp.float32)
        for i in range(ZCH):
            pltpu.sync_copy(zerov, dacc.at[pl.ds(s * RPT + i * CHUNK, CHUNK)])
        pltpu.sync_copy(dst_hbm.at[w], dstv)
        plsc.subcore_barrier()

        def body(k, carry):
            pltpu.sync_copy(onev, dacc.at[dstv.at[k]], add=True)
            return carry

        lax.fori_loop(0, EPWP // CHUNK, body, 0)
        plsc.subcore_barrier()
        pltpu.sync_copy(dacc.at[pl.ds(s * RPT, RPT)],
                        out_hbm.at[pl.ds(c * NACC + s * RPT, RPT)])

    return deg_kernel(dst_3d)


def _edge_pass(g, idx_hbm):
    @functools.partial(
        pl.kernel,
        out_type=jax.ShapeDtypeStruct((NC * NACC, D), jnp.float32),
        mesh=_sc_mesh(),
        scratch_types=[
            pltpu.VMEM((2 * CPC, CHUNK), jnp.int32),
            pltpu.VMEM((ECH, D), jnp.float32),
            pltpu.VMEM_SHARED((NACC, D), jnp.float32),
            pltpu.SemaphoreType.DMA,
            pltpu.SemaphoreType.DMA,
        ],
    )
    def edge_kernel(g_hbm, idx_h, out_hbm, idxb, rows, acc, gsem, ssem):
        c = lax.axis_index("c")
        s = lax.axis_index("s")
        w = c * NS + s

        def zero_row(r, carry):
            for j in range(D // 16):
                rows[r, pl.ds(j * 16, 16)] = jnp.zeros((16,), jnp.float32)
            return carry

        lax.fori_loop(0, ECH, zero_row, 0)
        base = s * RPT
        pltpu.sync_copy(rows, acc.at[pl.ds(base, ECH)])
        pltpu.sync_copy(rows, acc.at[pl.ds(base + ECH, ECH)])
        pltpu.sync_copy(rows.at[pl.ds(0, RPT - 2 * ECH)],
                        acc.at[pl.ds(base + 2 * ECH, RPT - 2 * ECH)])
        plsc.subcore_barrier()

        def body(k, carry):
            pltpu.sync_copy(idx_h.at[w * NCH + k], idxb)
            gathers = [
                pltpu.async_copy(
                    g_hbm.at[idxb.at[i]],
                    rows.at[pl.ds(i * CHUNK, CHUNK)], gsem)
                for i in range(CPC)
            ]
            for gth in gathers:
                gth.wait()
            scatters = [
                pltpu.async_copy(
                    rows.at[pl.ds(i * CHUNK, CHUNK)],
                    acc.at[idxb.at[CPC + i]], ssem, add=True)
                for i in range(CPC)
            ]
            for sct in scatters:
                sct.wait()
            return carry

        lax.fori_loop(0, NCH, body, 0)
        plsc.subcore_barrier()
        pltpu.sync_copy(acc.at[pl.ds(s * RPT, RPT)],
                        out_hbm.at[pl.ds(c * NACC + s * RPT, RPT)])

    return edge_kernel(g, idx_hbm)


def _tc1_body(x_ref, w_ref, dinv_ref, g_ref):
    h = jnp.dot(x_ref[...], w_ref[...], preferred_element_type=jnp.float32)
    g_ref[...] = h * dinv_ref[...]


def _tc2_body(acc_ref, g_ref, dinv_ref, b_ref, w_ref, o_ref):
    accsum = acc_ref[0:N, :] + acc_ref[NACC:NACC + N, :]
    hf = jax.nn.relu((accsum + g_ref[...]) * dinv_ref[...] + b_ref[...])
    h2 = jnp.dot(hf, w_ref[...], preferred_element_type=jnp.float32)
    o_ref[...] = h2 * dinv_ref[...]


def _tc3_body(acc_ref, g_ref, dinv_ref, b_ref, wo_ref, bo_ref, o_ref):
    accsum = acc_ref[0:N, :] + acc_ref[NACC:NACC + N, :]
    hf = jax.nn.relu((accsum + g_ref[...]) * dinv_ref[...] + b_ref[...])
    z = jnp.dot(hf, wo_ref[...], preferred_element_type=jnp.float32)
    o_ref[...] = jax.nn.sigmoid(z + bo_ref[...])


def kernel(x, edge_index, W1, b1, W2, b2, Wo, bo):
    ei = edge_index.astype(jnp.int32)
    pad = ((0, 0), (0, EPWP - EPW))
    src_p = jnp.pad(ei[0].reshape(NW, EPW), pad)
    dst_p = jnp.pad(ei[1].reshape(NW, EPW), pad, constant_values=JUNK)
    idx_hbm = jnp.concatenate(
        [src_p.reshape(NW, NCH, CPC, CHUNK),
         dst_p.reshape(NW, NCH, CPC, CHUNK)], axis=2,
    ).reshape(NW * NCH, 2 * CPC, CHUNK)
    dst_deg = dst_p.reshape(NW, EPWP // CHUNK, CHUNK)

    deg2 = _deg_pass(dst_deg)
    deg = deg2[:N] + deg2[NACC:NACC + N] + 1.0
    dinv = lax.rsqrt(deg)[:, None]

    g1 = pl.pallas_call(
        _tc1_body,
        out_shape=jax.ShapeDtypeStruct((N, D), jnp.float32),
    )(x, W1, dinv)

    acc1 = _edge_pass(g1, idx_hbm)

    g2 = pl.pallas_call(
        _tc2_body,
        out_shape=jax.ShapeDtypeStruct((N, D), jnp.float32),
    )(acc1, g1, dinv, b1.reshape(1, D), W2)

    acc2 = _edge_pass(g2, idx_hbm)

    out = pl.pallas_call(
        _tc3_body,
        out_shape=jax.ShapeDtypeStruct((N, 1), jnp.float32),
    )(acc2, g2, dinv, b2.reshape(1, D), Wo, bo.reshape(1, 1))
    return out


# speedup vs baseline: 1.4395x; 1.1633x over previous
import functools

import jax
import jax.numpy as jnp
from jax import lax
from jax.experimental import pallas as pl
from jax.experimental.pallas import tpu as pltpu
import jax.experimental.pallas.tpu_sc as plsc

N = 10000
D = 128
E = 320000
NC = 2
NS = 16
NW = NC * NS
EPW = E // NW
CHUNK = 128
NCH = -(-EPW // CHUNK)
EPWP = NCH * CHUNK
NACC = 10240
RPT = NACC // NS
ZCH = RPT // CHUNK
JUNK = N + 16


def _sc_mesh():
    return plsc.VectorSubcoreMesh(core_axis_name="c", subcore_axis_name="s")


def _deg_pass(dst_flat):
    @functools.partial(
        pl.kernel,
        out_type=jax.ShapeDtypeStruct((NC * NACC,), jnp.float32),
        mesh=_sc_mesh(),
        scratch_types=[
            pltpu.VMEM((CHUNK,), jnp.int32),
            pltpu.VMEM((CHUNK,), jnp.float32),
            pltpu.VMEM((CHUNK,), jnp.float32),
            pltpu.VMEM_SHARED((NACC,), jnp.float32),
        ],
    )
    def deg_kernel(dst_hbm, out_hbm, dstv, onev, zerov, dacc):
        c = lax.axis_index("c")
        s = lax.axis_index("s")
        w = c * NS + s
        for j in range(CHUNK // 16):
            onev[pl.ds(j * 16, 16)] = jnp.ones((16,), jnp.float32)
            zerov[pl.ds(j * 16, 16)] = jnp.zeros((16,), jnp.float32)
        for i in range(ZCH):
            pltpu.sync_copy(zerov, dacc.at[pl.ds(s * RPT + i * CHUNK, CHUNK)])
        plsc.subcore_barrier()
        ebase = w * EPWP

        def body(k, carry):
            b = ebase + k * CHUNK
            pltpu.sync_copy(dst_hbm.at[pl.ds(b, CHUNK)], dstv)
            pltpu.sync_copy(onev, dacc.at[dstv], add=True)
            return carry

        lax.fori_loop(0, NCH, body, 0)
        plsc.subcore_barrier()
        pltpu.sync_copy(dacc.at[pl.ds(s * RPT, RPT)],
                        out_hbm.at[pl.ds(c * NACC + s * RPT, RPT)])

    return deg_kernel(dst_flat)


def _edge_pass(g, src_flat, dst_flat):
    @functools.partial(
        pl.kernel,
        out_type=jax.ShapeDtypeStruct((NC * NACC, D), jnp.float32),
        mesh=_sc_mesh(),
        scratch_types=[
            pltpu.VMEM((CHUNK,), jnp.int32),
            pltpu.VMEM((CHUNK,), jnp.int32),
            pltpu.VMEM((CHUNK, D), jnp.float32),
            pltpu.VMEM_SHARED((NACC, D), jnp.float32),
            pltpu.SemaphoreType.DMA,
        ],
    )
    def edge_kernel(g_hbm, src_hbm, dst_hbm, out_hbm, srcv, dstv, rows, acc,
                    gsem):
        c = lax.axis_index("c")
        s = lax.axis_index("s")
        w = c * NS + s

        def zero_row(r, carry):
            for j in range(D // 16):
                rows[r, pl.ds(j * 16, 16)] = jnp.zeros((16,), jnp.float32)
            return carry

        lax.fori_loop(0, CHUNK, zero_row, 0)
        for i in range(ZCH):
            pltpu.sync_copy(rows, acc.at[pl.ds(s * RPT + i * CHUNK, CHUNK)])
        plsc.subcore_barrier()
        ebase = w * EPWP

        def body(k, carry):
            b = ebase + k * CHUNK
            pltpu.sync_copy(src_hbm.at[pl.ds(b, CHUNK)], srcv)
            pltpu.sync_copy(dst_hbm.at[pl.ds(b, CHUNK)], dstv)
            pltpu.async_copy(g_hbm.at[srcv], rows, gsem).wait()
            pltpu.sync_copy(rows, acc.at[dstv], add=True)
            return carry

        lax.fori_loop(0, NCH, body, 0)
        plsc.subcore_barrier()
        pltpu.sync_copy(acc.at[pl.ds(s * RPT, RPT)],
                        out_hbm.at[pl.ds(c * NACC + s * RPT, RPT)])

    return edge_kernel(g, src_flat, dst_flat)


def _tc1_body(x_ref, w_ref, dinv_ref, g_ref):
    h = jnp.dot(x_ref[...], w_ref[...], preferred_element_type=jnp.float32)
    g_ref[...] = h * dinv_ref[...]


def _tc2_body(acc_ref, g_ref, dinv_ref, b_ref, w_ref, o_ref):
    accsum = acc_ref[0:N, :] + acc_ref[NACC:NACC + N, :]
    hf = jax.nn.relu((accsum + g_ref[...]) * dinv_ref[...] + b_ref[...])
    h2 = jnp.dot(hf, w_ref[...], preferred_element_type=jnp.float32)
    o_ref[...] = h2 * dinv_ref[...]


def _tc3_body(acc_ref, g_ref, dinv_ref, b_ref, wo_ref, bo_ref, o_ref):
    accsum = acc_ref[0:N, :] + acc_ref[NACC:NACC + N, :]
    hf = jax.nn.relu((accsum + g_ref[...]) * dinv_ref[...] + b_ref[...])
    z = jnp.dot(hf, wo_ref[...], preferred_element_type=jnp.float32)
    o_ref[...] = jax.nn.sigmoid(z + bo_ref[...])


def kernel(x, edge_index, W1, b1, W2, b2, Wo, bo):
    ei = edge_index.astype(jnp.int32)
    pad = ((0, 0), (0, EPWP - EPW))
    src_flat = jnp.pad(ei[0].reshape(NW, EPW), pad).reshape(NW * EPWP)
    dst_flat = jnp.pad(ei[1].reshape(NW, EPW), pad,
                       constant_values=JUNK).reshape(NW * EPWP)

    deg2 = _deg_pass(dst_flat)
    deg = deg2[:N] + deg2[NACC:NACC + N] + 1.0
    dinv = lax.rsqrt(deg)[:, None]

    g1 = pl.pallas_call(
        _tc1_body,
        out_shape=jax.ShapeDtypeStruct((N, D), jnp.float32),
    )(x, W1, dinv)

    acc1 = _edge_pass(g1, src_flat, dst_flat)

    g2 = pl.pallas_call(
        _tc2_body,
        out_shape=jax.ShapeDtypeStruct((N, D), jnp.float32),
    )(acc1, g1, dinv, b1.reshape(1, D), W2)

    acc2 = _edge_pass(g2, src_flat, dst_flat)

    out = pl.pallas_call(
        _tc3_body,
        out_shape=jax.ShapeDtypeStruct((N, 1), jnp.float32),
    )(acc2, g2, dinv, b2.reshape(1, D), Wo, bo.reshape(1, 1))
    return out
